# baseline (speedup 1.0000x reference)
import functools

import jax
import jax.numpy as jnp
from jax import lax
from jax.experimental import pallas as pl
from jax.experimental.pallas import tpu as pltpu
from jax.experimental.pallas import tpu_sc as plsc

N = 10000
E = 320000
D = 128
NPAD = 10240
CHUNK = 64
NW = 32
EPAD = 327680
TOTAL_CH = EPAD // CHUNK
RING = 2
BLK = 8
NCH0 = 240
NCH1 = 80
RPT = NPAD // 16
RB = 1280
_F32 = jnp.float32

DEG_EPW = EPAD // NW


def _deg_body(row_hbm, out_hbm, idx_v, deg_v):
    cid = lax.axis_index("c")
    sid = lax.axis_index("s")
    wid = cid * 16 + sid
    zeros16 = jnp.zeros((16,), _F32)
    ones16 = jnp.ones((16,), _F32)

    def zbody(i, carry):
        deg_v[pl.ds(i * 16, 16)] = zeros16
        return carry

    lax.fori_loop(0, NPAD // 16, zbody, 0)
    pltpu.sync_copy(row_hbm.at[pl.ds(wid * DEG_EPW, DEG_EPW)], idx_v)

    def sbody(i, carry):
        idx = idx_v[pl.ds(i * 16, 16)]
        plsc.addupdate_scatter(deg_v, [idx], ones16)
        return carry

    lax.fori_loop(0, DEG_EPW // 16, sbody, 0)
    pltpu.sync_copy(deg_v, out_hbm.at[wid])


@functools.lru_cache(maxsize=None)
def _get_deg_call():
    return pl.kernel(
        _deg_body,
        out_type=jax.ShapeDtypeStruct((NW, NPAD), _F32),
        mesh=plsc.VectorSubcoreMesh(core_axis_name="c", subcore_axis_name="s"),
        compiler_params=pltpu.CompilerParams(needs_layout_passes=False),
        scratch_types=[
            pltpu.VMEM((DEG_EPW,), jnp.int32),
            pltpu.VMEM((NPAD,), _F32),
        ],
    )


def _make_agg(n_src):
    def body(*refs):
        u_refs = refs[:n_src]
        row_hbm, col_hbm, zeros_hbm = refs[n_src:n_src + 3]
        out_refs = refs[n_src + 3:2 * n_src + 3]
        scr = refs[2 * n_src + 3:]
        rs0, rs1, cs0, cs1 = scr[0:4]
        bufs = scr[4:4 + RING]
        acc = scr[4 + RING]
        isem0, isem1 = scr[5 + RING:7 + RING]
        gsems = scr[7 + RING:7 + RING + RING]
        cid = lax.axis_index("c")
        sid = lax.axis_index("s")
        rbase = sid * RPT
        nch_me = jnp.where(cid == 0, NCH0, NCH1)
        cbase = cid * 16 * NCH0 + sid * nch_me
        nblk2 = nch_me // (2 * BLK)

        def istart(k, rs, cs, isem):
            base = cbase + k * BLK
            pltpu.async_copy(row_hbm.at[pl.ds(base * CHUNK, BLK * CHUNK)],
                             rs, isem)
            pltpu.async_copy(col_hbm.at[pl.ds(base, BLK)], cs, isem)

        def iwait(k, rs, cs, isem):
            base = cbase + k * BLK
            pltpu.make_async_copy(
                row_hbm.at[pl.ds(base * CHUNK, BLK * CHUNK)], rs,
                isem).wait()
            pltpu.make_async_copy(col_hbm.at[pl.ds(base, BLK)], cs,
                                  isem).wait()

        def gstart(g, rs, b, buf, gsem):
            pltpu.async_copy(u_refs[g].at[rs.at[pl.ds(b * CHUNK, CHUNK)]],
                             buf, gsem)

        def gwait(g, rs, b, buf, gsem):
            pltpu.make_async_copy(
                u_refs[g].at[rs.at[pl.ds(b * CHUNK, CHUNK)]], buf,
                gsem).wait()

        def process(g, k, rs, cs, nk, nrs, ncs, nisem, next_pred):
            for b in range(BLK):
                la = b + RING - 1
                if la < BLK:
                    gstart(g, rs, la, bufs[la % RING], gsems[la % RING])
                else:
                    nb = la - BLK

                    def _ahead(nb=nb):
                        if nb == 0:
                            iwait(nk, nrs, ncs, nisem)
                        gstart(g, nrs, nb, bufs[nb % RING], gsems[nb % RING])

                    if next_pred is None:
                        _ahead()
                    else:
                        pl.when(next_pred)(_ahead)
                gwait(g, rs, b, bufs[b % RING], gsems[b % RING])
                pltpu.sync_copy(bufs[b % RING], acc.at[cs.at[b]], add=True)

        for g in range(n_src):
            istart(0, rs0, cs0, isem0)
            istart(1, rs1, cs1, isem1)
            pltpu.sync_copy(zeros_hbm.at[pl.ds(rbase, RPT)],
                            acc.at[pl.ds(rbase, RPT)])
            plsc.subcore_barrier()
            iwait(0, rs0, cs0, isem0)
            for b in range(RING - 1):
                gstart(g, rs0, b, bufs[b], gsems[b])

            def pbody(p, carry, g=g):
                k0 = 2 * p
                not_last = p < nblk2 - 1
                process(g, k0, rs0, cs0, k0 + 1, rs1, cs1, isem1, None)

                @pl.when(not_last)
                def _():
                    istart(k0 + 2, rs0, cs0, isem0)

                process(g, k0 + 1, rs1, cs1, k0 + 2, rs0, cs0, isem0,
                        not_last)

                @pl.when(not_last)
                def _():
                    istart(k0 + 3, rs1, cs1, isem1)

                return carry

            lax.fori_loop(0, nblk2, pbody, 0)
            plsc.subcore_barrier()
            pltpu.sync_copy(acc.at[pl.ds(rbase, RPT)],
                            out_refs[g].at[cid, pl.ds(rbase, RPT)])

    return pl.kernel(
        body,
        out_type=[jax.ShapeDtypeStruct((2, NPAD, D), _F32)] * n_src,
        mesh=plsc.VectorSubcoreMesh(core_axis_name="c", subcore_axis_name="s"),
        scratch_types=(
            [pltpu.VMEM((BLK * CHUNK,), jnp.int32)] * 2
            + [pltpu.VMEM((BLK, CHUNK), jnp.int32)] * 2
            + [pltpu.VMEM((CHUNK, D), _F32)] * RING
            + [pltpu.VMEM_SHARED((NPAD, D), _F32)]
            + [pltpu.SemaphoreType.DMA] * (2 + RING)
        ),
    )


_make_agg = functools.lru_cache(maxsize=None)(_make_agg)


def _prep_body(degp_ref, x_ref, h0_ref, h1_ref,
               dinv_ref, u0_ref, u1_ref, u2_ref):
    deg = jnp.sum(degp_ref[...], axis=0) + 1.0
    dinv = lax.rsqrt(deg)
    dinv_ref[...] = dinv[:, None]
    d2 = dinv[:, None]
    u0_ref[...] = d2 * x_ref[...]
    u1_ref[...] = d2 * h0_ref[...]
    u2_ref[...] = d2 * h1_ref[...]


_prep_call = pl.pallas_call(
    _prep_body,
    grid=(NPAD // RB,),
    in_specs=[
        pl.BlockSpec((NW, RB), lambda b: (0, b)),
        pl.BlockSpec((RB, D), lambda b: (b, 0)),
        pl.BlockSpec((RB, D), lambda b: (b, 0)),
        pl.BlockSpec((RB, D), lambda b: (b, 0)),
    ],
    out_specs=[
        pl.BlockSpec((RB, 1), lambda b: (b, 0)),
        pl.BlockSpec((RB, D), lambda b: (b, 0)),
        pl.BlockSpec((RB, D), lambda b: (b, 0)),
        pl.BlockSpec((RB, D), lambda b: (b, 0)),
    ],
    out_shape=[
        jax.ShapeDtypeStruct((NPAD, 1), _F32),
        jax.ShapeDtypeStruct((NPAD, D), _F32),
        jax.ShapeDtypeStruct((NPAD, D), _F32),
        jax.ShapeDtypeStruct((NPAD, D), _F32),
    ],
)


def _layer_body(sx_ref, sh_ref, ux_ref, uh_ref, dinv_ref, hi_ref, ci_ref,
                wx_ref, wh_ref, b_ref, hout_ref, cout_ref, un_ref):
    d2 = dinv_ref[...]
    aggx = d2 * (sx_ref[0] + sx_ref[1] + ux_ref[...])
    aggh = d2 * (sh_ref[0] + sh_ref[1] + uh_ref[...])
    z = (jnp.dot(aggx, wx_ref[...], preferred_element_type=_F32,
                 precision=lax.Precision.HIGHEST)
         + jnp.dot(aggh, wh_ref[...], preferred_element_type=_F32,
                   precision=lax.Precision.HIGHEST)
         + b_ref[...])
    ig = jax.nn.sigmoid(z[:, 0:D])
    fg = jax.nn.sigmoid(z[:, D:2 * D])
    og = jax.nn.sigmoid(z[:, 2 * D:3 * D])
    tg = jnp.tanh(z[:, 3 * D:4 * D])
    c_new = fg * hi_ref[...] + ig * tg
    h_new = og * jnp.tanh(ci_ref[...])
    hout_ref[...] = h_new
    cout_ref[...] = c_new
    un_ref[...] = d2 * h_new


_layer_call = pl.pallas_call(
    _layer_body,
    grid=(NPAD // RB,),
    in_specs=[
        pl.BlockSpec((2, RB, D), lambda b: (0, b, 0)),
        pl.BlockSpec((2, RB, D), lambda b: (0, b, 0)),
        pl.BlockSpec((RB, D), lambda b: (b, 0)),
        pl.BlockSpec((RB, D), lambda b: (b, 0)),
        pl.BlockSpec((RB, 1), lambda b: (b, 0)),
        pl.BlockSpec((RB, D), lambda b: (b, 0)),
        pl.BlockSpec((RB, D), lambda b: (b, 0)),
        pl.BlockSpec((D, 4 * D), lambda b: (0, 0)),
        pl.BlockSpec((D, 4 * D), lambda b: (0, 0)),
        pl.BlockSpec((1, 4 * D), lambda b: (0, 0)),
    ],
    out_specs=[
        pl.BlockSpec((RB, D), lambda b: (b, 0)),
        pl.BlockSpec((RB, D), lambda b: (b, 0)),
        pl.BlockSpec((RB, D), lambda b: (b, 0)),
    ],
    out_shape=[
        jax.ShapeDtypeStruct((NPAD, D), _F32),
        jax.ShapeDtypeStruct((NPAD, D), _F32),
        jax.ShapeDtypeStruct((NPAD, D), _F32),
    ],
)


def kernel(x, edge_idx, h, c, Wx, Wh, bx, bh):
    row = edge_idx[0].astype(jnp.int32)
    col = edge_idx[1].astype(jnp.int32)
    npad_rows = NPAD - N
    epad = EPAD - E
    dummy = jnp.full((epad,), NPAD - 1, jnp.int32)
    row_p = jnp.concatenate([row, dummy])
    col_r = jnp.concatenate([col, dummy]).reshape(TOTAL_CH, CHUNK)

    pad2 = ((0, npad_rows), (0, 0))
    xp = jnp.pad(x, pad2)
    h0p = jnp.pad(h[0], pad2)
    h1p = jnp.pad(h[1], pad2)
    c0p = jnp.pad(c[0], pad2)
    c1p = jnp.pad(c[1], pad2)
    zeros = jnp.zeros((NPAD, D), _F32)

    wx0 = Wx[0].transpose(1, 0, 2).reshape(D, 4 * D)
    wx1 = Wx[1].transpose(1, 0, 2).reshape(D, 4 * D)
    wh0 = Wh[0].transpose(1, 0, 2).reshape(D, 4 * D)
    wh1 = Wh[1].transpose(1, 0, 2).reshape(D, 4 * D)
    b0 = (bx[0] + bh[0]).reshape(1, 4 * D)
    b1 = (bx[1] + bh[1]).reshape(1, 4 * D)

    deg_part = _get_deg_call()(row_p)
    dinv, u0, u1, u2 = _prep_call(deg_part, xp, h0p, h1p)
    s0, s1 = _make_agg(2)(u0, u1, row_p, col_r, zeros)
    h0n, c0n, unext = _layer_call(s0, s1, u0, u1, dinv, h0p, c0p,
                                  wx0, wh0, b0)
    (s2,) = _make_agg(1)(u2, row_p, col_r, zeros)
    (s3,) = _make_agg(1)(unext, row_p, col_r, zeros)
    h1n, c1n, _ = _layer_call(s3, s2, unext, u2, dinv, h1p, c1p,
                              wx1, wh1, b1)
    h_out = jnp.stack([h0n[:N], h1n[:N]], axis=0)
    c_out = jnp.stack([c0n[:N], c1n[:N]], axis=0)
    return (h_out, c_out)


# speedup vs baseline: 1.1326x; 1.1326x over previous
import functools

import jax
import jax.numpy as jnp
from jax import lax
from jax.experimental import pallas as pl
from jax.experimental.pallas import tpu as pltpu
from jax.experimental.pallas import tpu_sc as plsc

N = 10000
E = 320000
D = 128
NPAD = 10240
CHUNK = 64
NW = 32
EPAD = 327680
TOTAL_CH = EPAD // CHUNK
RING = 2
BLK = 8
NCH0 = 240
NCH1 = 80
RPT = NPAD // 16
RB = 1280
_F32 = jnp.float32

DEG_EPW = EPAD // NW


def _deg_body(row_hbm, out_hbm, idx_v, deg_v):
    cid = lax.axis_index("c")
    sid = lax.axis_index("s")
    wid = cid * 16 + sid
    zeros16 = jnp.zeros((16,), _F32)
    ones16 = jnp.ones((16,), _F32)

    def zbody(i, carry):
        deg_v[pl.ds(i * 16, 16)] = zeros16
        return carry

    lax.fori_loop(0, NPAD // 16, zbody, 0)
    pltpu.sync_copy(row_hbm.at[pl.ds(wid * DEG_EPW, DEG_EPW)], idx_v)

    def sbody(i, carry):
        idx = idx_v[pl.ds(i * 16, 16)]
        plsc.addupdate_scatter(deg_v, [idx], ones16)
        return carry

    lax.fori_loop(0, DEG_EPW // 16, sbody, 0)
    pltpu.sync_copy(deg_v, out_hbm.at[wid])


@functools.lru_cache(maxsize=None)
def _get_deg_call():
    return pl.kernel(
        _deg_body,
        out_type=jax.ShapeDtypeStruct((NW, NPAD), _F32),
        mesh=plsc.VectorSubcoreMesh(core_axis_name="c", subcore_axis_name="s"),
        compiler_params=pltpu.CompilerParams(needs_layout_passes=False),
        scratch_types=[
            pltpu.VMEM((DEG_EPW,), jnp.int32),
            pltpu.VMEM((NPAD,), _F32),
        ],
    )


def _make_agg(n_src):
    def body(*refs):
        u_refs = refs[:n_src]
        row_hbm, col_hbm, zeros_hbm = refs[n_src:n_src + 3]
        out_refs = refs[n_src + 3:2 * n_src + 3]
        scr = refs[2 * n_src + 3:]
        rs0, rs1, cs0, cs1 = scr[0:4]
        bufs = scr[4:4 + RING]
        acc = scr[4 + RING]
        isem0, isem1 = scr[5 + RING:7 + RING]
        gsems = scr[7 + RING:7 + RING + RING]
        cid = lax.axis_index("c")
        sid = lax.axis_index("s")
        rbase = sid * RPT
        nch_me = jnp.where(cid == 0, NCH0, NCH1)
        cbase = cid * 16 * NCH0 + sid * nch_me
        nblk2 = nch_me // (2 * BLK)

        def istart(k, rs, cs, isem):
            base = cbase + k * BLK
            pltpu.async_copy(row_hbm.at[pl.ds(base * CHUNK, BLK * CHUNK)],
                             rs, isem)
            pltpu.async_copy(col_hbm.at[pl.ds(base, BLK)], cs, isem)

        def iwait(k, rs, cs, isem):
            base = cbase + k * BLK
            pltpu.make_async_copy(
                row_hbm.at[pl.ds(base * CHUNK, BLK * CHUNK)], rs,
                isem).wait()
            pltpu.make_async_copy(col_hbm.at[pl.ds(base, BLK)], cs,
                                  isem).wait()

        def gstart(g, rs, b, buf, gsem):
            pltpu.async_copy(u_refs[g].at[rs.at[pl.ds(b * CHUNK, CHUNK)]],
                             buf, gsem)

        def gwait(g, rs, b, buf, gsem):
            pltpu.make_async_copy(
                u_refs[g].at[rs.at[pl.ds(b * CHUNK, CHUNK)]], buf,
                gsem).wait()

        def process(g, k, rs, cs, nk, nrs, ncs, nisem, next_pred):
            for b in range(BLK):
                la = b + RING - 1
                if la < BLK:
                    gstart(g, rs, la, bufs[la % RING], gsems[la % RING])
                else:
                    nb = la - BLK

                    def _ahead(nb=nb):
                        if nb == 0:
                            iwait(nk, nrs, ncs, nisem)
                        gstart(g, nrs, nb, bufs[nb % RING], gsems[nb % RING])

                    if next_pred is None:
                        _ahead()
                    else:
                        pl.when(next_pred)(_ahead)
                gwait(g, rs, b, bufs[b % RING], gsems[b % RING])
                pltpu.sync_copy(bufs[b % RING], acc.at[cs.at[b]], add=True)

        for g in range(n_src):
            istart(0, rs0, cs0, isem0)
            istart(1, rs1, cs1, isem1)
            pltpu.sync_copy(zeros_hbm.at[pl.ds(rbase, RPT)],
                            acc.at[pl.ds(rbase, RPT)])
            plsc.subcore_barrier()
            iwait(0, rs0, cs0, isem0)
            for b in range(RING - 1):
                gstart(g, rs0, b, bufs[b], gsems[b])

            def pbody(p, carry, g=g):
                k0 = 2 * p
                not_last = p < nblk2 - 1
                process(g, k0, rs0, cs0, k0 + 1, rs1, cs1, isem1, None)

                @pl.when(not_last)
                def _():
                    istart(k0 + 2, rs0, cs0, isem0)

                process(g, k0 + 1, rs1, cs1, k0 + 2, rs0, cs0, isem0,
                        not_last)

                @pl.when(not_last)
                def _():
                    istart(k0 + 3, rs1, cs1, isem1)

                return carry

            lax.fori_loop(0, nblk2, pbody, 0)
            plsc.subcore_barrier()
            pltpu.sync_copy(acc.at[pl.ds(rbase, RPT)],
                            out_refs[g].at[cid, pl.ds(rbase, RPT)])

    return pl.kernel(
        body,
        out_type=[jax.ShapeDtypeStruct((2, NPAD, D), _F32)] * n_src,
        mesh=plsc.VectorSubcoreMesh(core_axis_name="c", subcore_axis_name="s"),
        scratch_types=(
            [pltpu.VMEM((BLK * CHUNK,), jnp.int32)] * 2
            + [pltpu.VMEM((BLK, CHUNK), jnp.int32)] * 2
            + [pltpu.VMEM((CHUNK, D), _F32)] * RING
            + [pltpu.VMEM_SHARED((NPAD, D), _F32)]
            + [pltpu.SemaphoreType.DMA] * (2 + RING)
        ),
    )


_make_agg = functools.lru_cache(maxsize=None)(_make_agg)


def _prep_body(degp_ref, x_ref, h0_ref, h1_ref,
               dinv_ref, u0_ref, u1_ref, u2_ref):
    deg = jnp.sum(degp_ref[...], axis=0) + 1.0
    dinv = lax.rsqrt(deg)
    dinv_ref[...] = dinv[:, None]
    d2 = dinv[:, None]
    u0_ref[...] = d2 * x_ref[...]
    u1_ref[...] = d2 * h0_ref[...]
    u2_ref[...] = d2 * h1_ref[...]


_prep_call = pl.pallas_call(
    _prep_body,
    grid=(NPAD // RB,),
    in_specs=[
        pl.BlockSpec((NW, RB), lambda b: (0, b)),
        pl.BlockSpec((RB, D), lambda b: (b, 0)),
        pl.BlockSpec((RB, D), lambda b: (b, 0)),
        pl.BlockSpec((RB, D), lambda b: (b, 0)),
    ],
    out_specs=[
        pl.BlockSpec((RB, 1), lambda b: (b, 0)),
        pl.BlockSpec((RB, D), lambda b: (b, 0)),
        pl.BlockSpec((RB, D), lambda b: (b, 0)),
        pl.BlockSpec((RB, D), lambda b: (b, 0)),
    ],
    out_shape=[
        jax.ShapeDtypeStruct((NPAD, 1), _F32),
        jax.ShapeDtypeStruct((NPAD, D), _F32),
        jax.ShapeDtypeStruct((NPAD, D), _F32),
        jax.ShapeDtypeStruct((NPAD, D), _F32),
    ],
)


def _layer_body(sx_ref, sh_ref, ux_ref, uh_ref, dinv_ref, hi_ref, ci_ref,
                wx_ref, wh_ref, b_ref, hout_ref, cout_ref, un_ref):
    d2 = dinv_ref[...]
    aggx = d2 * (sx_ref[0] + sx_ref[1] + ux_ref[...])
    aggh = d2 * (sh_ref[0] + sh_ref[1] + uh_ref[...])
    z = (jnp.dot(aggx, wx_ref[...], preferred_element_type=_F32,
                 precision=lax.Precision.HIGHEST)
         + jnp.dot(aggh, wh_ref[...], preferred_element_type=_F32,
                   precision=lax.Precision.HIGHEST)
         + b_ref[...])
    ig = jax.nn.sigmoid(z[:, 0:D])
    fg = jax.nn.sigmoid(z[:, D:2 * D])
    og = jax.nn.sigmoid(z[:, 2 * D:3 * D])
    tg = jnp.tanh(z[:, 3 * D:4 * D])
    c_new = fg * hi_ref[...] + ig * tg
    h_new = og * jnp.tanh(ci_ref[...])
    hout_ref[...] = h_new
    cout_ref[...] = c_new
    un_ref[...] = d2 * h_new


_layer_call = pl.pallas_call(
    _layer_body,
    grid=(NPAD // RB,),
    in_specs=[
        pl.BlockSpec((2, RB, D), lambda b: (0, b, 0)),
        pl.BlockSpec((2, RB, D), lambda b: (0, b, 0)),
        pl.BlockSpec((RB, D), lambda b: (b, 0)),
        pl.BlockSpec((RB, D), lambda b: (b, 0)),
        pl.BlockSpec((RB, 1), lambda b: (b, 0)),
        pl.BlockSpec((RB, D), lambda b: (b, 0)),
        pl.BlockSpec((RB, D), lambda b: (b, 0)),
        pl.BlockSpec((D, 4 * D), lambda b: (0, 0)),
        pl.BlockSpec((D, 4 * D), lambda b: (0, 0)),
        pl.BlockSpec((1, 4 * D), lambda b: (0, 0)),
    ],
    out_specs=[
        pl.BlockSpec((RB, D), lambda b: (b, 0)),
        pl.BlockSpec((RB, D), lambda b: (b, 0)),
        pl.BlockSpec((RB, D), lambda b: (b, 0)),
    ],
    out_shape=[
        jax.ShapeDtypeStruct((NPAD, D), _F32),
        jax.ShapeDtypeStruct((NPAD, D), _F32),
        jax.ShapeDtypeStruct((NPAD, D), _F32),
    ],
)


def kernel(x, edge_idx, h, c, Wx, Wh, bx, bh):
    row = edge_idx[0].astype(jnp.int32)
    col = edge_idx[1].astype(jnp.int32)
    npad_rows = NPAD - N
    epad = EPAD - E
    dummy = jnp.full((epad,), NPAD - 1, jnp.int32)
    row_p = jnp.concatenate([row, dummy])
    col_r = jnp.concatenate([col, dummy]).reshape(TOTAL_CH, CHUNK)

    pad2 = ((0, npad_rows), (0, 0))
    xp = jnp.pad(x, pad2)
    h0p = jnp.pad(h[0], pad2)
    h1p = jnp.pad(h[1], pad2)
    c0p = jnp.pad(c[0], pad2)
    c1p = jnp.pad(c[1], pad2)
    zeros = jnp.zeros((NPAD, D), _F32)

    wx0 = Wx[0].transpose(1, 0, 2).reshape(D, 4 * D)
    wx1 = Wx[1].transpose(1, 0, 2).reshape(D, 4 * D)
    wh0 = Wh[0].transpose(1, 0, 2).reshape(D, 4 * D)
    wh1 = Wh[1].transpose(1, 0, 2).reshape(D, 4 * D)
    b0 = (bx[0] + bh[0]).reshape(1, 4 * D)
    b1 = (bx[1] + bh[1]).reshape(1, 4 * D)

    deg_part = _get_deg_call()(row_p)
    dinv, u0, u1, u2 = _prep_call(deg_part, xp, h0p, h1p)
    s0, s1, s2 = _make_agg(3)(u0, u1, u2, row_p, col_r, zeros)
    h0n, c0n, unext = _layer_call(s0, s1, u0, u1, dinv, h0p, c0p,
                                  wx0, wh0, b0)
    (s3,) = _make_agg(1)(unext, row_p, col_r, zeros)
    h1n, c1n, _ = _layer_call(s3, s2, unext, u2, dinv, h1p, c1p,
                              wx1, wh1, b1)
    h_out = jnp.stack([h0n[:N], h1n[:N]], axis=0)
    c_out = jnp.stack([c0n[:N], c1n[:N]], axis=0)
    return (h_out, c_out)


# speedup vs baseline: 1.1377x; 1.0044x over previous
import functools

import jax
import jax.numpy as jnp
from jax import lax
from jax.experimental import pallas as pl
from jax.experimental.pallas import tpu as pltpu
from jax.experimental.pallas import tpu_sc as plsc

N = 10000
E = 320000
D = 128
NPAD = 10240
CHUNK = 64
NW = 32
EPAD = 327680
TOTAL_CH = EPAD // CHUNK
RING = 2
BLK = 8
NCH0 = 256
NCH1 = 64
RPT = NPAD // 16
RB = 1280
_F32 = jnp.float32

DEG_EPW = EPAD // NW


def _deg_body(row_hbm, out_hbm, idx_v, deg_v):
    cid = lax.axis_index("c")
    sid = lax.axis_index("s")
    wid = cid * 16 + sid
    zeros16 = jnp.zeros((16,), _F32)
    ones16 = jnp.ones((16,), _F32)

    def zbody(i, carry):
        deg_v[pl.ds(i * 16, 16)] = zeros16
        return carry

    lax.fori_loop(0, NPAD // 16, zbody, 0)
    pltpu.sync_copy(row_hbm.at[pl.ds(wid * DEG_EPW, DEG_EPW)], idx_v)

    def sbody(i, carry):
        idx = idx_v[pl.ds(i * 16, 16)]
        plsc.addupdate_scatter(deg_v, [idx], ones16)
        return carry

    lax.fori_loop(0, DEG_EPW // 16, sbody, 0)
    pltpu.sync_copy(deg_v, out_hbm.at[wid])


@functools.lru_cache(maxsize=None)
def _get_deg_call():
    return pl.kernel(
        _deg_body,
        out_type=jax.ShapeDtypeStruct((NW, NPAD), _F32),
        mesh=plsc.VectorSubcoreMesh(core_axis_name="c", subcore_axis_name="s"),
        compiler_params=pltpu.CompilerParams(needs_layout_passes=False),
        scratch_types=[
            pltpu.VMEM((DEG_EPW,), jnp.int32),
            pltpu.VMEM((NPAD,), _F32),
        ],
    )


def _make_agg(n_src):
    def body(*refs):
        u_refs = refs[:n_src]
        row_hbm, col_hbm, zeros_hbm = refs[n_src:n_src + 3]
        out_refs = refs[n_src + 3:2 * n_src + 3]
        scr = refs[2 * n_src + 3:]
        rs0, rs1, cs0, cs1 = scr[0:4]
        bufs = scr[4:4 + RING]
        acc = scr[4 + RING]
        isem0, isem1 = scr[5 + RING:7 + RING]
        gsems = scr[7 + RING:7 + RING + RING]
        cid = lax.axis_index("c")
        sid = lax.axis_index("s")
        rbase = sid * RPT
        nch_me = jnp.where(cid == 0, NCH0, NCH1)
        cbase = cid * 16 * NCH0 + sid * nch_me
        nblk2 = nch_me // (2 * BLK)

        def istart(k, rs, cs, isem):
            base = cbase + k * BLK
            pltpu.async_copy(row_hbm.at[pl.ds(base * CHUNK, BLK * CHUNK)],
                             rs, isem)
            pltpu.async_copy(col_hbm.at[pl.ds(base, BLK)], cs, isem)

        def iwait(k, rs, cs, isem):
            base = cbase + k * BLK
            pltpu.make_async_copy(
                row_hbm.at[pl.ds(base * CHUNK, BLK * CHUNK)], rs,
                isem).wait()
            pltpu.make_async_copy(col_hbm.at[pl.ds(base, BLK)], cs,
                                  isem).wait()

        def gstart(g, rs, b, buf, gsem):
            pltpu.async_copy(u_refs[g].at[rs.at[pl.ds(b * CHUNK, CHUNK)]],
                             buf, gsem)

        def gwait(g, rs, b, buf, gsem):
            pltpu.make_async_copy(
                u_refs[g].at[rs.at[pl.ds(b * CHUNK, CHUNK)]], buf,
                gsem).wait()

        def process(g, k, rs, cs, nk, nrs, ncs, nisem, next_pred):
            for b in range(BLK):
                la = b + RING - 1
                if la < BLK:
                    gstart(g, rs, la, bufs[la % RING], gsems[la % RING])
                else:
                    nb = la - BLK

                    def _ahead(nb=nb):
                        if nb == 0:
                            iwait(nk, nrs, ncs, nisem)
                        gstart(g, nrs, nb, bufs[nb % RING], gsems[nb % RING])

                    if next_pred is None:
                        _ahead()
                    else:
                        pl.when(next_pred)(_ahead)
                gwait(g, rs, b, bufs[b % RING], gsems[b % RING])
                pltpu.sync_copy(bufs[b % RING], acc.at[cs.at[b]], add=True)

        for g in range(n_src):
            istart(0, rs0, cs0, isem0)
            istart(1, rs1, cs1, isem1)
            pltpu.sync_copy(zeros_hbm.at[pl.ds(rbase, RPT)],
                            acc.at[pl.ds(rbase, RPT)])
            plsc.subcore_barrier()
            iwait(0, rs0, cs0, isem0)
            for b in range(RING - 1):
                gstart(g, rs0, b, bufs[b], gsems[b])

            def pbody(p, carry, g=g):
                k0 = 2 * p
                not_last = p < nblk2 - 1
                process(g, k0, rs0, cs0, k0 + 1, rs1, cs1, isem1, None)

                @pl.when(not_last)
                def _():
                    istart(k0 + 2, rs0, cs0, isem0)

                process(g, k0 + 1, rs1, cs1, k0 + 2, rs0, cs0, isem0,
                        not_last)

                @pl.when(not_last)
                def _():
                    istart(k0 + 3, rs1, cs1, isem1)

                return carry

            lax.fori_loop(0, nblk2, pbody, 0)
            plsc.subcore_barrier()
            pltpu.sync_copy(acc.at[pl.ds(rbase, RPT)],
                            out_refs[g].at[cid, pl.ds(rbase, RPT)])

    return pl.kernel(
        body,
        out_type=[jax.ShapeDtypeStruct((2, NPAD, D), _F32)] * n_src,
        mesh=plsc.VectorSubcoreMesh(core_axis_name="c", subcore_axis_name="s"),
        scratch_types=(
            [pltpu.VMEM((BLK * CHUNK,), jnp.int32)] * 2
            + [pltpu.VMEM((BLK, CHUNK), jnp.int32)] * 2
            + [pltpu.VMEM((CHUNK, D), _F32)] * RING
            + [pltpu.VMEM_SHARED((NPAD, D), _F32)]
            + [pltpu.SemaphoreType.DMA] * (2 + RING)
        ),
    )


_make_agg = functools.lru_cache(maxsize=None)(_make_agg)


def _prep_body(degp_ref, x_ref, h0_ref, h1_ref,
               dinv_ref, u0_ref, u1_ref, u2_ref):
    deg = jnp.sum(degp_ref[...], axis=0) + 1.0
    dinv = lax.rsqrt(deg)
    dinv_ref[...] = dinv[:, None]
    d2 = dinv[:, None]
    u0_ref[...] = d2 * x_ref[...]
    u1_ref[...] = d2 * h0_ref[...]
    u2_ref[...] = d2 * h1_ref[...]


_prep_call = pl.pallas_call(
    _prep_body,
    grid=(NPAD // RB,),
    in_specs=[
        pl.BlockSpec((NW, RB), lambda b: (0, b)),
        pl.BlockSpec((RB, D), lambda b: (b, 0)),
        pl.BlockSpec((RB, D), lambda b: (b, 0)),
        pl.BlockSpec((RB, D), lambda b: (b, 0)),
    ],
    out_specs=[
        pl.BlockSpec((RB, 1), lambda b: (b, 0)),
        pl.BlockSpec((RB, D), lambda b: (b, 0)),
        pl.BlockSpec((RB, D), lambda b: (b, 0)),
        pl.BlockSpec((RB, D), lambda b: (b, 0)),
    ],
    out_shape=[
        jax.ShapeDtypeStruct((NPAD, 1), _F32),
        jax.ShapeDtypeStruct((NPAD, D), _F32),
        jax.ShapeDtypeStruct((NPAD, D), _F32),
        jax.ShapeDtypeStruct((NPAD, D), _F32),
    ],
)


def _layer_body(sx_ref, sh_ref, ux_ref, uh_ref, dinv_ref, hi_ref, ci_ref,
                wx_ref, wh_ref, b_ref, hout_ref, cout_ref, un_ref):
    d2 = dinv_ref[...]
    aggx = d2 * (sx_ref[0] + sx_ref[1] + ux_ref[...])
    aggh = d2 * (sh_ref[0] + sh_ref[1] + uh_ref[...])
    z = (jnp.dot(aggx, wx_ref[...], preferred_element_type=_F32,
                 precision=lax.Precision.HIGHEST)
         + jnp.dot(aggh, wh_ref[...], preferred_element_type=_F32,
                   precision=lax.Precision.HIGHEST)
         + b_ref[...])
    ig = jax.nn.sigmoid(z[:, 0:D])
    fg = jax.nn.sigmoid(z[:, D:2 * D])
    og = jax.nn.sigmoid(z[:, 2 * D:3 * D])
    tg = jnp.tanh(z[:, 3 * D:4 * D])
    c_new = fg * hi_ref[...] + ig * tg
    h_new = og * jnp.tanh(ci_ref[...])
    hout_ref[...] = h_new
    cout_ref[...] = c_new
    un_ref[...] = d2 * h_new


_layer_call = pl.pallas_call(
    _layer_body,
    grid=(NPAD // RB,),
    in_specs=[
        pl.BlockSpec((2, RB, D), lambda b: (0, b, 0)),
        pl.BlockSpec((2, RB, D), lambda b: (0, b, 0)),
        pl.BlockSpec((RB, D), lambda b: (b, 0)),
        pl.BlockSpec((RB, D), lambda b: (b, 0)),
        pl.BlockSpec((RB, 1), lambda b: (b, 0)),
        pl.BlockSpec((RB, D), lambda b: (b, 0)),
        pl.BlockSpec((RB, D), lambda b: (b, 0)),
        pl.BlockSpec((D, 4 * D), lambda b: (0, 0)),
        pl.BlockSpec((D, 4 * D), lambda b: (0, 0)),
        pl.BlockSpec((1, 4 * D), lambda b: (0, 0)),
    ],
    out_specs=[
        pl.BlockSpec((RB, D), lambda b: (b, 0)),
        pl.BlockSpec((RB, D), lambda b: (b, 0)),
        pl.BlockSpec((RB, D), lambda b: (b, 0)),
    ],
    out_shape=[
        jax.ShapeDtypeStruct((NPAD, D), _F32),
        jax.ShapeDtypeStruct((NPAD, D), _F32),
        jax.ShapeDtypeStruct((NPAD, D), _F32),
    ],
)


def kernel(x, edge_idx, h, c, Wx, Wh, bx, bh):
    row = edge_idx[0].astype(jnp.int32)
    col = edge_idx[1].astype(jnp.int32)
    npad_rows = NPAD - N
    epad = EPAD - E
    dummy = jnp.full((epad,), NPAD - 1, jnp.int32)
    row_p = jnp.concatenate([row, dummy])
    col_r = jnp.concatenate([col, dummy]).reshape(TOTAL_CH, CHUNK)

    pad2 = ((0, npad_rows), (0, 0))
    xp = jnp.pad(x, pad2)
    h0p = jnp.pad(h[0], pad2)
    h1p = jnp.pad(h[1], pad2)
    c0p = jnp.pad(c[0], pad2)
    c1p = jnp.pad(c[1], pad2)
    zeros = jnp.zeros((NPAD, D), _F32)

    wx0 = Wx[0].transpose(1, 0, 2).reshape(D, 4 * D)
    wx1 = Wx[1].transpose(1, 0, 2).reshape(D, 4 * D)
    wh0 = Wh[0].transpose(1, 0, 2).reshape(D, 4 * D)
    wh1 = Wh[1].transpose(1, 0, 2).reshape(D, 4 * D)
    b0 = (bx[0] + bh[0]).reshape(1, 4 * D)
    b1 = (bx[1] + bh[1]).reshape(1, 4 * D)

    deg_part = _get_deg_call()(row_p)
    dinv, u0, u1, u2 = _prep_call(deg_part, xp, h0p, h1p)
    s0, s1, s2 = _make_agg(3)(u0, u1, u2, row_p, col_r, zeros)
    h0n, c0n, unext = _layer_call(s0, s1, u0, u1, dinv, h0p, c0p,
                                  wx0, wh0, b0)
    (s3,) = _make_agg(1)(unext, row_p, col_r, zeros)
    h1n, c1n, _ = _layer_call(s3, s2, unext, u2, dinv, h1p, c1p,
                              wx1, wh1, b1)
    h_out = jnp.stack([h0n[:N], h1n[:N]], axis=0)
    c_out = jnp.stack([c0n[:N], c1n[:N]], axis=0)
    return (h_out, c_out)


# speedup vs baseline: 1.1612x; 1.0207x over previous
import functools

import jax
import jax.numpy as jnp
from jax import lax
from jax.experimental import pallas as pl
from jax.experimental.pallas import tpu as pltpu
from jax.experimental.pallas import tpu_sc as plsc

N = 10000
E = 320000
D = 128
NPAD = 10240
CHUNK = 64
NW = 32
EPAD = 327680
TOTAL_CH = EPAD // CHUNK
RING = 2
BLK = 8
NCH0 = 272
NCH1 = 48
RPT = NPAD // 16
RB = 1280
_F32 = jnp.float32

DEG_EPW = EPAD // NW


def _deg_body(row_hbm, out_hbm, idx_v, deg_v):
    cid = lax.axis_index("c")
    sid = lax.axis_index("s")
    wid = cid * 16 + sid
    zeros16 = jnp.zeros((16,), _F32)
    ones16 = jnp.ones((16,), _F32)

    def zbody(i, carry):
        deg_v[pl.ds(i * 16, 16)] = zeros16
        return carry

    lax.fori_loop(0, NPAD // 16, zbody, 0)
    pltpu.sync_copy(row_hbm.at[pl.ds(wid * DEG_EPW, DEG_EPW)], idx_v)

    def sbody(i, carry):
        idx = idx_v[pl.ds(i * 16, 16)]
        plsc.addupdate_scatter(deg_v, [idx], ones16)
        return carry

    lax.fori_loop(0, DEG_EPW // 16, sbody, 0)
    pltpu.sync_copy(deg_v, out_hbm.at[wid])


@functools.lru_cache(maxsize=None)
def _get_deg_call():
    return pl.kernel(
        _deg_body,
        out_type=jax.ShapeDtypeStruct((NW, NPAD), _F32),
        mesh=plsc.VectorSubcoreMesh(core_axis_name="c", subcore_axis_name="s"),
        compiler_params=pltpu.CompilerParams(needs_layout_passes=False),
        scratch_types=[
            pltpu.VMEM((DEG_EPW,), jnp.int32),
            pltpu.VMEM((NPAD,), _F32),
        ],
    )


def _make_agg(n_src):
    def body(*refs):
        u_refs = refs[:n_src]
        row_hbm, col_hbm, zeros_hbm = refs[n_src:n_src + 3]
        out_refs = refs[n_src + 3:2 * n_src + 3]
        scr = refs[2 * n_src + 3:]
        rs0, rs1, cs0, cs1 = scr[0:4]
        bufs = scr[4:4 + RING]
        acc = scr[4 + RING]
        isem0, isem1 = scr[5 + RING:7 + RING]
        gsems = scr[7 + RING:7 + RING + RING]
        cid = lax.axis_index("c")
        sid = lax.axis_index("s")
        rbase = sid * RPT
        nch_me = jnp.where(cid == 0, NCH0, NCH1)
        cbase = cid * 16 * NCH0 + sid * nch_me
        nblk2 = nch_me // (2 * BLK)

        def istart(k, rs, cs, isem):
            base = cbase + k * BLK
            pltpu.async_copy(row_hbm.at[pl.ds(base * CHUNK, BLK * CHUNK)],
                             rs, isem)
            pltpu.async_copy(col_hbm.at[pl.ds(base, BLK)], cs, isem)

        def iwait(k, rs, cs, isem):
            base = cbase + k * BLK
            pltpu.make_async_copy(
                row_hbm.at[pl.ds(base * CHUNK, BLK * CHUNK)], rs,
                isem).wait()
            pltpu.make_async_copy(col_hbm.at[pl.ds(base, BLK)], cs,
                                  isem).wait()

        def gstart(g, rs, b, buf, gsem):
            pltpu.async_copy(u_refs[g].at[rs.at[pl.ds(b * CHUNK, CHUNK)]],
                             buf, gsem)

        def gwait(g, rs, b, buf, gsem):
            pltpu.make_async_copy(
                u_refs[g].at[rs.at[pl.ds(b * CHUNK, CHUNK)]], buf,
                gsem).wait()

        def process(g, k, rs, cs, nk, nrs, ncs, nisem, next_pred):
            for b in range(BLK):
                la = b + RING - 1
                if la < BLK:
                    gstart(g, rs, la, bufs[la % RING], gsems[la % RING])
                else:
                    nb = la - BLK

                    def _ahead(nb=nb):
                        if nb == 0:
                            iwait(nk, nrs, ncs, nisem)
                        gstart(g, nrs, nb, bufs[nb % RING], gsems[nb % RING])

                    if next_pred is None:
                        _ahead()
                    else:
                        pl.when(next_pred)(_ahead)
                gwait(g, rs, b, bufs[b % RING], gsems[b % RING])
                pltpu.sync_copy(bufs[b % RING], acc.at[cs.at[b]], add=True)

        for g in range(n_src):
            istart(0, rs0, cs0, isem0)
            istart(1, rs1, cs1, isem1)
            pltpu.sync_copy(zeros_hbm.at[pl.ds(rbase, RPT)],
                            acc.at[pl.ds(rbase, RPT)])
            plsc.subcore_barrier()
            iwait(0, rs0, cs0, isem0)
            for b in range(RING - 1):
                gstart(g, rs0, b, bufs[b], gsems[b])

            def pbody(p, carry, g=g):
                k0 = 2 * p
                not_last = p < nblk2 - 1
                process(g, k0, rs0, cs0, k0 + 1, rs1, cs1, isem1, None)

                @pl.when(not_last)
                def _():
                    istart(k0 + 2, rs0, cs0, isem0)

                process(g, k0 + 1, rs1, cs1, k0 + 2, rs0, cs0, isem0,
                        not_last)

                @pl.when(not_last)
                def _():
                    istart(k0 + 3, rs1, cs1, isem1)

                return carry

            lax.fori_loop(0, nblk2, pbody, 0)
            plsc.subcore_barrier()
            pltpu.sync_copy(acc.at[pl.ds(rbase, RPT)],
                            out_refs[g].at[cid, pl.ds(rbase, RPT)])

    return pl.kernel(
        body,
        out_type=[jax.ShapeDtypeStruct((2, NPAD, D), _F32)] * n_src,
        mesh=plsc.VectorSubcoreMesh(core_axis_name="c", subcore_axis_name="s"),
        scratch_types=(
            [pltpu.VMEM((BLK * CHUNK,), jnp.int32)] * 2
            + [pltpu.VMEM((BLK, CHUNK), jnp.int32)] * 2
            + [pltpu.VMEM((CHUNK, D), _F32)] * RING
            + [pltpu.VMEM_SHARED((NPAD, D), _F32)]
            + [pltpu.SemaphoreType.DMA] * (2 + RING)
        ),
    )


_make_agg = functools.lru_cache(maxsize=None)(_make_agg)


def _prep_body(degp_ref, x_ref, h0_ref, h1_ref,
               dinv_ref, u0_ref, u1_ref, u2_ref):
    deg = jnp.sum(degp_ref[...], axis=0) + 1.0
    dinv = lax.rsqrt(deg)
    dinv_ref[...] = dinv[:, None]
    d2 = dinv[:, None]
    u0_ref[...] = d2 * x_ref[...]
    u1_ref[...] = d2 * h0_ref[...]
    u2_ref[...] = d2 * h1_ref[...]


_prep_call = pl.pallas_call(
    _prep_body,
    grid=(NPAD // RB,),
    in_specs=[
        pl.BlockSpec((NW, RB), lambda b: (0, b)),
        pl.BlockSpec((RB, D), lambda b: (b, 0)),
        pl.BlockSpec((RB, D), lambda b: (b, 0)),
        pl.BlockSpec((RB, D), lambda b: (b, 0)),
    ],
    out_specs=[
        pl.BlockSpec((RB, 1), lambda b: (b, 0)),
        pl.BlockSpec((RB, D), lambda b: (b, 0)),
        pl.BlockSpec((RB, D), lambda b: (b, 0)),
        pl.BlockSpec((RB, D), lambda b: (b, 0)),
    ],
    out_shape=[
        jax.ShapeDtypeStruct((NPAD, 1), _F32),
        jax.ShapeDtypeStruct((NPAD, D), _F32),
        jax.ShapeDtypeStruct((NPAD, D), _F32),
        jax.ShapeDtypeStruct((NPAD, D), _F32),
    ],
)


def _layer_body(sx_ref, sh_ref, ux_ref, uh_ref, dinv_ref, hi_ref, ci_ref,
                wx_ref, wh_ref, b_ref, hout_ref, cout_ref, un_ref):
    d2 = dinv_ref[...]
    aggx = d2 * (sx_ref[0] + sx_ref[1] + ux_ref[...])
    aggh = d2 * (sh_ref[0] + sh_ref[1] + uh_ref[...])
    z = (jnp.dot(aggx, wx_ref[...], preferred_element_type=_F32,
                 precision=lax.Precision.HIGHEST)
         + jnp.dot(aggh, wh_ref[...], preferred_element_type=_F32,
                   precision=lax.Precision.HIGHEST)
         + b_ref[...])
    ig = jax.nn.sigmoid(z[:, 0:D])
    fg = jax.nn.sigmoid(z[:, D:2 * D])
    og = jax.nn.sigmoid(z[:, 2 * D:3 * D])
    tg = jnp.tanh(z[:, 3 * D:4 * D])
    c_new = fg * hi_ref[...] + ig * tg
    h_new = og * jnp.tanh(ci_ref[...])
    hout_ref[...] = h_new
    cout_ref[...] = c_new
    un_ref[...] = d2 * h_new


_layer_call = pl.pallas_call(
    _layer_body,
    grid=(NPAD // RB,),
    in_specs=[
        pl.BlockSpec((2, RB, D), lambda b: (0, b, 0)),
        pl.BlockSpec((2, RB, D), lambda b: (0, b, 0)),
        pl.BlockSpec((RB, D), lambda b: (b, 0)),
        pl.BlockSpec((RB, D), lambda b: (b, 0)),
        pl.BlockSpec((RB, 1), lambda b: (b, 0)),
        pl.BlockSpec((RB, D), lambda b: (b, 0)),
        pl.BlockSpec((RB, D), lambda b: (b, 0)),
        pl.BlockSpec((D, 4 * D), lambda b: (0, 0)),
        pl.BlockSpec((D, 4 * D), lambda b: (0, 0)),
        pl.BlockSpec((1, 4 * D), lambda b: (0, 0)),
    ],
    out_specs=[
        pl.BlockSpec((RB, D), lambda b: (b, 0)),
        pl.BlockSpec((RB, D), lambda b: (b, 0)),
        pl.BlockSpec((RB, D), lambda b: (b, 0)),
    ],
    out_shape=[
        jax.ShapeDtypeStruct((NPAD, D), _F32),
        jax.ShapeDtypeStruct((NPAD, D), _F32),
        jax.ShapeDtypeStruct((NPAD, D), _F32),
    ],
)


def kernel(x, edge_idx, h, c, Wx, Wh, bx, bh):
    row = edge_idx[0].astype(jnp.int32)
    col = edge_idx[1].astype(jnp.int32)
    npad_rows = NPAD - N
    epad = EPAD - E
    dummy = jnp.full((epad,), NPAD - 1, jnp.int32)
    row_p = jnp.concatenate([row, dummy])
    col_r = jnp.concatenate([col, dummy]).reshape(TOTAL_CH, CHUNK)

    pad2 = ((0, npad_rows), (0, 0))
    xp = jnp.pad(x, pad2)
    h0p = jnp.pad(h[0], pad2)
    h1p = jnp.pad(h[1], pad2)
    c0p = jnp.pad(c[0], pad2)
    c1p = jnp.pad(c[1], pad2)
    zeros = jnp.zeros((NPAD, D), _F32)

    wx0 = Wx[0].transpose(1, 0, 2).reshape(D, 4 * D)
    wx1 = Wx[1].transpose(1, 0, 2).reshape(D, 4 * D)
    wh0 = Wh[0].transpose(1, 0, 2).reshape(D, 4 * D)
    wh1 = Wh[1].transpose(1, 0, 2).reshape(D, 4 * D)
    b0 = (bx[0] + bh[0]).reshape(1, 4 * D)
    b1 = (bx[1] + bh[1]).reshape(1, 4 * D)

    deg_part = _get_deg_call()(row_p)
    dinv, u0, u1, u2 = _prep_call(deg_part, xp, h0p, h1p)
    s0, s1, s2 = _make_agg(3)(u0, u1, u2, row_p, col_r, zeros)
    h0n, c0n, unext = _layer_call(s0, s1, u0, u1, dinv, h0p, c0p,
                                  wx0, wh0, b0)
    (s3,) = _make_agg(1)(unext, row_p, col_r, zeros)
    h1n, c1n, _ = _layer_call(s3, s2, unext, u2, dinv, h1p, c1p,
                              wx1, wh1, b1)
    h_out = jnp.stack([h0n[:N], h1n[:N]], axis=0)
    c_out = jnp.stack([c0n[:N], c1n[:N]], axis=0)
    return (h_out, c_out)


# speedup vs baseline: 1.1820x; 1.0180x over previous
import functools

import jax
import jax.numpy as jnp
from jax import lax
from jax.experimental import pallas as pl
from jax.experimental.pallas import tpu as pltpu
from jax.experimental.pallas import tpu_sc as plsc

N = 10000
E = 320000
D = 128
NPAD = 10240
CHUNK = 64
NW = 32
EPAD = 327680
TOTAL_CH = EPAD // CHUNK
RING = 2
BLK = 8
NCH0 = 288
NCH1 = 32
RPT = NPAD // 16
RB = 1280
_F32 = jnp.float32

DEG_EPW = EPAD // NW


def _deg_body(row_hbm, out_hbm, idx_v, deg_v):
    cid = lax.axis_index("c")
    sid = lax.axis_index("s")
    wid = cid * 16 + sid
    zeros16 = jnp.zeros((16,), _F32)
    ones16 = jnp.ones((16,), _F32)

    def zbody(i, carry):
        deg_v[pl.ds(i * 16, 16)] = zeros16
        return carry

    lax.fori_loop(0, NPAD // 16, zbody, 0)
    pltpu.sync_copy(row_hbm.at[pl.ds(wid * DEG_EPW, DEG_EPW)], idx_v)

    def sbody(i, carry):
        idx = idx_v[pl.ds(i * 16, 16)]
        plsc.addupdate_scatter(deg_v, [idx], ones16)
        return carry

    lax.fori_loop(0, DEG_EPW // 16, sbody, 0)
    pltpu.sync_copy(deg_v, out_hbm.at[wid])


@functools.lru_cache(maxsize=None)
def _get_deg_call():
    return pl.kernel(
        _deg_body,
        out_type=jax.ShapeDtypeStruct((NW, NPAD), _F32),
        mesh=plsc.VectorSubcoreMesh(core_axis_name="c", subcore_axis_name="s"),
        compiler_params=pltpu.CompilerParams(needs_layout_passes=False),
        scratch_types=[
            pltpu.VMEM((DEG_EPW,), jnp.int32),
            pltpu.VMEM((NPAD,), _F32),
        ],
    )


def _make_agg(n_src):
    def body(*refs):
        u_refs = refs[:n_src]
        row_hbm, col_hbm, zeros_hbm = refs[n_src:n_src + 3]
        out_refs = refs[n_src + 3:2 * n_src + 3]
        scr = refs[2 * n_src + 3:]
        rs0, rs1, cs0, cs1 = scr[0:4]
        bufs = scr[4:4 + RING]
        acc = scr[4 + RING]
        isem0, isem1 = scr[5 + RING:7 + RING]
        gsems = scr[7 + RING:7 + RING + RING]
        cid = lax.axis_index("c")
        sid = lax.axis_index("s")
        rbase = sid * RPT
        nch_me = jnp.where(cid == 0, NCH0, NCH1)
        cbase = cid * 16 * NCH0 + sid * nch_me
        nblk2 = nch_me // (2 * BLK)

        def istart(k, rs, cs, isem):
            base = cbase + k * BLK
            pltpu.async_copy(row_hbm.at[pl.ds(base * CHUNK, BLK * CHUNK)],
                             rs, isem)
            pltpu.async_copy(col_hbm.at[pl.ds(base, BLK)], cs, isem)

        def iwait(k, rs, cs, isem):
            base = cbase + k * BLK
            pltpu.make_async_copy(
                row_hbm.at[pl.ds(base * CHUNK, BLK * CHUNK)], rs,
                isem).wait()
            pltpu.make_async_copy(col_hbm.at[pl.ds(base, BLK)], cs,
                                  isem).wait()

        def gstart(g, rs, b, buf, gsem):
            pltpu.async_copy(u_refs[g].at[rs.at[pl.ds(b * CHUNK, CHUNK)]],
                             buf, gsem)

        def gwait(g, rs, b, buf, gsem):
            pltpu.make_async_copy(
                u_refs[g].at[rs.at[pl.ds(b * CHUNK, CHUNK)]], buf,
                gsem).wait()

        def process(g, k, rs, cs, nk, nrs, ncs, nisem, next_pred):
            for b in range(BLK):
                la = b + RING - 1
                if la < BLK:
                    gstart(g, rs, la, bufs[la % RING], gsems[la % RING])
                else:
                    nb = la - BLK

                    def _ahead(nb=nb):
                        if nb == 0:
                            iwait(nk, nrs, ncs, nisem)
                        gstart(g, nrs, nb, bufs[nb % RING], gsems[nb % RING])

                    if next_pred is None:
                        _ahead()
                    else:
                        pl.when(next_pred)(_ahead)
                gwait(g, rs, b, bufs[b % RING], gsems[b % RING])
                pltpu.sync_copy(bufs[b % RING], acc.at[cs.at[b]], add=True)

        for g in range(n_src):
            istart(0, rs0, cs0, isem0)
            istart(1, rs1, cs1, isem1)
            pltpu.sync_copy(zeros_hbm.at[pl.ds(rbase, RPT)],
                            acc.at[pl.ds(rbase, RPT)])
            plsc.subcore_barrier()
            iwait(0, rs0, cs0, isem0)
            for b in range(RING - 1):
                gstart(g, rs0, b, bufs[b], gsems[b])

            def pbody(p, carry, g=g):
                k0 = 2 * p
                not_last = p < nblk2 - 1
                process(g, k0, rs0, cs0, k0 + 1, rs1, cs1, isem1, None)

                @pl.when(not_last)
                def _():
                    istart(k0 + 2, rs0, cs0, isem0)

                process(g, k0 + 1, rs1, cs1, k0 + 2, rs0, cs0, isem0,
                        not_last)

                @pl.when(not_last)
                def _():
                    istart(k0 + 3, rs1, cs1, isem1)

                return carry

            lax.fori_loop(0, nblk2, pbody, 0)
            plsc.subcore_barrier()
            pltpu.sync_copy(acc.at[pl.ds(rbase, RPT)],
                            out_refs[g].at[cid, pl.ds(rbase, RPT)])

    return pl.kernel(
        body,
        out_type=[jax.ShapeDtypeStruct((2, NPAD, D), _F32)] * n_src,
        mesh=plsc.VectorSubcoreMesh(core_axis_name="c", subcore_axis_name="s"),
        scratch_types=(
            [pltpu.VMEM((BLK * CHUNK,), jnp.int32)] * 2
            + [pltpu.VMEM((BLK, CHUNK), jnp.int32)] * 2
            + [pltpu.VMEM((CHUNK, D), _F32)] * RING
            + [pltpu.VMEM_SHARED((NPAD, D), _F32)]
            + [pltpu.SemaphoreType.DMA] * (2 + RING)
        ),
    )


_make_agg = functools.lru_cache(maxsize=None)(_make_agg)


def _prep_body(degp_ref, x_ref, h0_ref, h1_ref,
               dinv_ref, u0_ref, u1_ref, u2_ref):
    deg = jnp.sum(degp_ref[...], axis=0) + 1.0
    dinv = lax.rsqrt(deg)
    dinv_ref[...] = dinv[:, None]
    d2 = dinv[:, None]
    u0_ref[...] = d2 * x_ref[...]
    u1_ref[...] = d2 * h0_ref[...]
    u2_ref[...] = d2 * h1_ref[...]


_prep_call = pl.pallas_call(
    _prep_body,
    grid=(NPAD // RB,),
    in_specs=[
        pl.BlockSpec((NW, RB), lambda b: (0, b)),
        pl.BlockSpec((RB, D), lambda b: (b, 0)),
        pl.BlockSpec((RB, D), lambda b: (b, 0)),
        pl.BlockSpec((RB, D), lambda b: (b, 0)),
    ],
    out_specs=[
        pl.BlockSpec((RB, 1), lambda b: (b, 0)),
        pl.BlockSpec((RB, D), lambda b: (b, 0)),
        pl.BlockSpec((RB, D), lambda b: (b, 0)),
        pl.BlockSpec((RB, D), lambda b: (b, 0)),
    ],
    out_shape=[
        jax.ShapeDtypeStruct((NPAD, 1), _F32),
        jax.ShapeDtypeStruct((NPAD, D), _F32),
        jax.ShapeDtypeStruct((NPAD, D), _F32),
        jax.ShapeDtypeStruct((NPAD, D), _F32),
    ],
)


def _layer_body(sx_ref, sh_ref, ux_ref, uh_ref, dinv_ref, hi_ref, ci_ref,
                wx_ref, wh_ref, b_ref, hout_ref, cout_ref, un_ref):
    d2 = dinv_ref[...]
    aggx = d2 * (sx_ref[0] + sx_ref[1] + ux_ref[...])
    aggh = d2 * (sh_ref[0] + sh_ref[1] + uh_ref[...])
    z = (jnp.dot(aggx, wx_ref[...], preferred_element_type=_F32,
                 precision=lax.Precision.HIGHEST)
         + jnp.dot(aggh, wh_ref[...], preferred_element_type=_F32,
                   precision=lax.Precision.HIGHEST)
         + b_ref[...])
    ig = jax.nn.sigmoid(z[:, 0:D])
    fg = jax.nn.sigmoid(z[:, D:2 * D])
    og = jax.nn.sigmoid(z[:, 2 * D:3 * D])
    tg = jnp.tanh(z[:, 3 * D:4 * D])
    c_new = fg * hi_ref[...] + ig * tg
    h_new = og * jnp.tanh(ci_ref[...])
    hout_ref[...] = h_new
    cout_ref[...] = c_new
    un_ref[...] = d2 * h_new


_layer_call = pl.pallas_call(
    _layer_body,
    grid=(NPAD // RB,),
    in_specs=[
        pl.BlockSpec((2, RB, D), lambda b: (0, b, 0)),
        pl.BlockSpec((2, RB, D), lambda b: (0, b, 0)),
        pl.BlockSpec((RB, D), lambda b: (b, 0)),
        pl.BlockSpec((RB, D), lambda b: (b, 0)),
        pl.BlockSpec((RB, 1), lambda b: (b, 0)),
        pl.BlockSpec((RB, D), lambda b: (b, 0)),
        pl.BlockSpec((RB, D), lambda b: (b, 0)),
        pl.BlockSpec((D, 4 * D), lambda b: (0, 0)),
        pl.BlockSpec((D, 4 * D), lambda b: (0, 0)),
        pl.BlockSpec((1, 4 * D), lambda b: (0, 0)),
    ],
    out_specs=[
        pl.BlockSpec((RB, D), lambda b: (b, 0)),
        pl.BlockSpec((RB, D), lambda b: (b, 0)),
        pl.BlockSpec((RB, D), lambda b: (b, 0)),
    ],
    out_shape=[
        jax.ShapeDtypeStruct((NPAD, D), _F32),
        jax.ShapeDtypeStruct((NPAD, D), _F32),
        jax.ShapeDtypeStruct((NPAD, D), _F32),
    ],
)


def kernel(x, edge_idx, h, c, Wx, Wh, bx, bh):
    row = edge_idx[0].astype(jnp.int32)
    col = edge_idx[1].astype(jnp.int32)
    npad_rows = NPAD - N
    epad = EPAD - E
    dummy = jnp.full((epad,), NPAD - 1, jnp.int32)
    row_p = jnp.concatenate([row, dummy])
    col_r = jnp.concatenate([col, dummy]).reshape(TOTAL_CH, CHUNK)

    pad2 = ((0, npad_rows), (0, 0))
    xp = jnp.pad(x, pad2)
    h0p = jnp.pad(h[0], pad2)
    h1p = jnp.pad(h[1], pad2)
    c0p = jnp.pad(c[0], pad2)
    c1p = jnp.pad(c[1], pad2)
    zeros = jnp.zeros((NPAD, D), _F32)

    wx0 = Wx[0].transpose(1, 0, 2).reshape(D, 4 * D)
    wx1 = Wx[1].transpose(1, 0, 2).reshape(D, 4 * D)
    wh0 = Wh[0].transpose(1, 0, 2).reshape(D, 4 * D)
    wh1 = Wh[1].transpose(1, 0, 2).reshape(D, 4 * D)
    b0 = (bx[0] + bh[0]).reshape(1, 4 * D)
    b1 = (bx[1] + bh[1]).reshape(1, 4 * D)

    deg_part = _get_deg_call()(row_p)
    dinv, u0, u1, u2 = _prep_call(deg_part, xp, h0p, h1p)
    s0, s1, s2 = _make_agg(3)(u0, u1, u2, row_p, col_r, zeros)
    h0n, c0n, unext = _layer_call(s0, s1, u0, u1, dinv, h0p, c0p,
                                  wx0, wh0, b0)
    (s3,) = _make_agg(1)(unext, row_p, col_r, zeros)
    h1n, c1n, _ = _layer_call(s3, s2, unext, u2, dinv, h1p, c1p,
                              wx1, wh1, b1)
    h_out = jnp.stack([h0n[:N], h1n[:N]], axis=0)
    c_out = jnp.stack([c0n[:N], c1n[:N]], axis=0)
    return (h_out, c_out)
